# Initial kernel scaffold; baseline (speedup 1.0000x reference)
import jax
import jax.numpy as jnp
from jax.experimental import pallas as pl


def kernel(x63, x60, w1, b1, w2, gamma, beta):
    raise NotImplementedError("write your pallas kernel here")


# speedup vs baseline: 2.2941x; 2.2941x over previous
import jax
import jax.numpy as jnp
from jax.experimental import pallas as pl
from jax.experimental.pallas import tpu as pltpu

BN_EPS = 1e-3
F32 = jnp.float32


def _se_conv_kernel(x63_ref, w1t_ref, b1_ref, w2_ref, x_ref,
                    y_ref, sum_ref, ssq_ref):
    s = jax.nn.sigmoid(
        jnp.dot(x63_ref[0], w1t_ref[...], preferred_element_type=F32)
        + b1_ref[...])
    w2s = w2_ref[...] * s
    y = jnp.dot(w2s, x_ref[0], preferred_element_type=F32)
    y_ref[0] = y
    sum_ref[0] = jnp.sum(y, axis=1, keepdims=True)
    ssq_ref[0] = jnp.sum(y * y, axis=1, keepdims=True)


def _bn_apply_kernel(y_ref, scale_ref, shift_ref, out_ref):
    out_ref[0] = y_ref[0] * scale_ref[0] + shift_ref[0]


def kernel(x63, x60, w1, b1, w2, gamma, beta):
    N, C_mid, H, W = x60.shape
    C_se = x63.shape[1]
    C_out = w2.shape[0]
    hw = H * W

    x = x60.astype(F32).reshape(N, C_mid, hw)

    se_pad = 128 if C_se <= 128 else ((C_se + 127) // 128) * 128
    x63_p = jnp.pad(x63.astype(F32).reshape(N, 1, C_se),
                    ((0, 0), (0, 0), (0, se_pad - C_se)))
    w1t_p = jnp.pad(w1.astype(F32).T, ((0, se_pad - C_se), (0, 0)))
    b1_row = b1.astype(F32).reshape(1, C_mid)
    w2_m = w2.astype(F32)

    y, psum, pssq = pl.pallas_call(
        _se_conv_kernel,
        grid=(N,),
        in_specs=[
            pl.BlockSpec((1, 1, se_pad), lambda b: (b, 0, 0)),
            pl.BlockSpec((se_pad, C_mid), lambda b: (0, 0)),
            pl.BlockSpec((1, C_mid), lambda b: (0, 0)),
            pl.BlockSpec((C_out, C_mid), lambda b: (0, 0)),
            pl.BlockSpec((1, C_mid, hw), lambda b: (b, 0, 0)),
        ],
        out_specs=[
            pl.BlockSpec((1, C_out, hw), lambda b: (b, 0, 0)),
            pl.BlockSpec((1, C_out, 1), lambda b: (b, 0, 0)),
            pl.BlockSpec((1, C_out, 1), lambda b: (b, 0, 0)),
        ],
        out_shape=[
            jax.ShapeDtypeStruct((N, C_out, hw), F32),
            jax.ShapeDtypeStruct((N, C_out, 1), F32),
            jax.ShapeDtypeStruct((N, C_out, 1), F32),
        ],
        compiler_params=pltpu.CompilerParams(
            dimension_semantics=("parallel",)),
    )(x63_p, w1t_p, b1_row, w2_m, x)

    n_elems = jnp.asarray(N * hw, F32)
    mean = jnp.sum(psum, axis=0) / n_elems
    var = jnp.maximum(jnp.sum(pssq, axis=0) / n_elems - mean * mean, 0.0)
    inv = jax.lax.rsqrt(var + BN_EPS)
    scale = (gamma.astype(F32).reshape(C_out, 1) * inv).reshape(1, C_out, 1)
    shift = (beta.astype(F32).reshape(C_out, 1)
             - mean * gamma.astype(F32).reshape(C_out, 1) * inv
             ).reshape(1, C_out, 1)

    out = pl.pallas_call(
        _bn_apply_kernel,
        grid=(N,),
        in_specs=[
            pl.BlockSpec((1, C_out, hw), lambda b: (b, 0, 0)),
            pl.BlockSpec((1, C_out, 1), lambda b: (0, 0, 0)),
            pl.BlockSpec((1, C_out, 1), lambda b: (0, 0, 0)),
        ],
        out_specs=pl.BlockSpec((1, C_out, hw), lambda b: (b, 0, 0)),
        out_shape=jax.ShapeDtypeStruct((N, C_out, hw), F32),
        compiler_params=pltpu.CompilerParams(
            dimension_semantics=("parallel",)),
    )(y, scale, shift)

    return out.reshape(N, C_out, H, W)


# speedup vs baseline: 2.3096x; 1.0068x over previous
import jax
import jax.numpy as jnp
from jax.experimental import pallas as pl
from jax.experimental.pallas import tpu as pltpu

BN_EPS = 1e-3
F32 = jnp.float32


def _se_conv_kernel(x63_ref, w1t_ref, b1_ref, w2_ref, x_ref,
                    y_ref, sum_ref, ssq_ref):
    b = pl.program_id(0)

    s = jax.nn.sigmoid(
        jnp.dot(x63_ref[b], w1t_ref[...], preferred_element_type=F32)
        + b1_ref[...])
    w2s = w2_ref[...] * s
    y = jnp.dot(w2s, x_ref[0], preferred_element_type=F32)
    y_ref[0] = y

    @pl.when(b == 0)
    def _init():
        sum_ref[...] = jnp.zeros_like(sum_ref)
        ssq_ref[...] = jnp.zeros_like(ssq_ref)

    sum_ref[0] += jnp.sum(y, axis=1, keepdims=True)
    ssq_ref[0] += jnp.sum(y * y, axis=1, keepdims=True)


def _bn_apply_kernel(y_ref, scale_ref, shift_ref, out_ref):
    out_ref[0] = y_ref[0] * scale_ref[0] + shift_ref[0]


def kernel(x63, x60, w1, b1, w2, gamma, beta):
    N, C_mid, H, W = x60.shape
    C_se = x63.shape[1]
    C_out = w2.shape[0]
    hw = H * W

    x = x60.astype(F32).reshape(N, C_mid, hw)

    se_pad = 128 if C_se <= 128 else ((C_se + 127) // 128) * 128
    x63_p = jnp.pad(x63.astype(F32).reshape(N, 1, C_se),
                    ((0, 0), (0, 0), (0, se_pad - C_se)))
    w1t_p = jnp.pad(w1.astype(F32).T, ((0, se_pad - C_se), (0, 0)))
    b1_row = b1.astype(F32).reshape(1, C_mid)
    w2_m = w2.astype(F32)

    y, psum, pssq = pl.pallas_call(
        _se_conv_kernel,
        grid=(N,),
        in_specs=[
            pl.BlockSpec((N, 1, se_pad), lambda b: (0, 0, 0)),
            pl.BlockSpec((se_pad, C_mid), lambda b: (0, 0)),
            pl.BlockSpec((1, C_mid), lambda b: (0, 0)),
            pl.BlockSpec((C_out, C_mid), lambda b: (0, 0)),
            pl.BlockSpec((1, C_mid, hw), lambda b: (b, 0, 0)),
        ],
        out_specs=[
            pl.BlockSpec((1, C_out, hw), lambda b: (b, 0, 0)),
            pl.BlockSpec((1, C_out, 1), lambda b: (0, 0, 0)),
            pl.BlockSpec((1, C_out, 1), lambda b: (0, 0, 0)),
        ],
        out_shape=[
            jax.ShapeDtypeStruct((N, C_out, hw), F32),
            jax.ShapeDtypeStruct((1, C_out, 1), F32),
            jax.ShapeDtypeStruct((1, C_out, 1), F32),
        ],
        compiler_params=pltpu.CompilerParams(
            dimension_semantics=("arbitrary",)),
    )(x63_p, w1t_p, b1_row, w2_m, x)

    n_elems = jnp.asarray(N * hw, F32)
    mean = jnp.sum(psum, axis=0) / n_elems
    var = jnp.maximum(jnp.sum(pssq, axis=0) / n_elems - mean * mean, 0.0)
    inv = jax.lax.rsqrt(var + BN_EPS)
    scale = (gamma.astype(F32).reshape(C_out, 1) * inv).reshape(1, C_out, 1)
    shift = (beta.astype(F32).reshape(C_out, 1)
             - mean * gamma.astype(F32).reshape(C_out, 1) * inv
             ).reshape(1, C_out, 1)

    out = pl.pallas_call(
        _bn_apply_kernel,
        grid=(N,),
        in_specs=[
            pl.BlockSpec((1, C_out, hw), lambda b: (b, 0, 0)),
            pl.BlockSpec((1, C_out, 1), lambda b: (0, 0, 0)),
            pl.BlockSpec((1, C_out, 1), lambda b: (0, 0, 0)),
        ],
        out_specs=pl.BlockSpec((1, C_out, hw), lambda b: (b, 0, 0)),
        out_shape=jax.ShapeDtypeStruct((N, C_out, hw), F32),
        compiler_params=pltpu.CompilerParams(
            dimension_semantics=("parallel",)),
    )(y, scale, shift)

    return out.reshape(N, C_out, H, W)
